# baseline (speedup 1.0000x reference)
import jax
import jax.numpy as jnp
from jax import lax
from jax.experimental import pallas as pl
from jax.experimental.pallas import tpu as pltpu
from jax.experimental.pallas import tpu_sc as plsc

_BATCH = 4
_SEQ = 4096
_K = 2
_TOKENS = _BATCH * _SEQ
_NC = 2
_NS = 16
_NW = _NC * _NS
_T_PER_W = _TOKENS // _NW
_L = 16
_GROUPS = _T_PER_W // _L
_CHUNK = 128
_CH_PER_W = _T_PER_W * _K // _CHUNK


def _router_body(ids_hbm, table_hbm, out_hbm, ids_v, idx_v, gout_v, sem):
    wid = lax.axis_index("s") * _NC + lax.axis_index("c")
    pltpu.sync_copy(ids_hbm.at[pl.ds(wid * _T_PER_W, _T_PER_W)], ids_v)
    lanes2 = lax.iota(jnp.int32, _L) * 2
    for g in range(_GROUPS):
        v = ids_v[pl.ds(g * _L, _L)]
        a = v * 2
        row = idx_v.at[g // 4]
        col = lanes2 + (32 * (g % 4))
        plsc.store_scatter(row, [col], a)
        plsc.store_scatter(row, [col + 1], a + 1)
    copies = [
        pltpu.async_copy(table_hbm.at[idx_v.at[c]], gout_v.at[c], sem)
        for c in range(_CH_PER_W)
    ]
    for c in copies:
        c.wait()
    pltpu.sync_copy(gout_v, out_hbm.at[pl.ds(wid * _CH_PER_W, _CH_PER_W)])


@jax.jit
def _route(ids, table_flat):
    mesh = plsc.VectorSubcoreMesh(
        core_axis_name="c", subcore_axis_name="s", num_cores=_NC,
        num_subcores=_NS,
    )
    call = pl.kernel(
        _router_body,
        out_type=jax.ShapeDtypeStruct((_NW * _CH_PER_W, _CHUNK), jnp.int32),
        mesh=mesh,
        scratch_types=[
            pltpu.VMEM((_T_PER_W,), jnp.int32),
            pltpu.VMEM((_CH_PER_W, _CHUNK), jnp.int32),
            pltpu.VMEM((_CH_PER_W, _CHUNK), jnp.int32),
            pltpu.SemaphoreType.DMA,
        ],
        compiler_params=pltpu.CompilerParams(
            use_tc_tiling_on_sc=False, needs_layout_passes=False,
        ),
    )
    return call(ids, table_flat)


def kernel(input, hash_table):
    ids = input.reshape(_TOKENS).astype(jnp.int32)
    table_flat = hash_table.reshape(-1)
    out = _route(ids, table_flat)
    return out.reshape(_BATCH, _SEQ, _K)


# speedup vs baseline: 2.9314x; 2.9314x over previous
import jax
import jax.numpy as jnp
from jax import lax
from jax.experimental import pallas as pl
from jax.experimental.pallas import tpu as pltpu
from jax.experimental.pallas import tpu_sc as plsc

_VOCAB = 50257
_BATCH = 4
_SEQ = 4096
_K = 2
_NC = 2
_NS = 16
_NW = _NC * _NS
_L = 16
_CHUNK = 128
_NSB = _SEQ // _CHUNK
_BLOCKS_PER_W = _BATCH * _NSB // _NW


def _router_body(ids_hbm, table_hbm, out_hbm, ids_v, idx1_v, gout_v, sem):
    wid = lax.axis_index("s") * _NC + lax.axis_index("c")
    b = wid // (_NSB // _BLOCKS_PER_W)
    sbase = (wid % (_NSB // _BLOCKS_PER_W)) * _BLOCKS_PER_W
    for j in range(_BLOCKS_PER_W):
        pltpu.sync_copy(ids_hbm.at[sbase + j, b], ids_v.at[j])
    for j in range(_BLOCKS_PER_W):
        for g in range(_CHUNK // _L):
            sl = pl.ds(g * _L, _L)
            idx1_v[j, sl] = ids_v[j, sl] + _VOCAB
    copies = []
    for j in range(_BLOCKS_PER_W):
        copies.append(
            pltpu.async_copy(table_hbm.at[ids_v.at[j]], gout_v.at[j, 0], sem)
        )
        copies.append(
            pltpu.async_copy(table_hbm.at[idx1_v.at[j]], gout_v.at[j, 1], sem)
        )
    for c in copies:
        c.wait()
    pltpu.sync_copy(gout_v, out_hbm.at[b, pl.ds(sbase, _BLOCKS_PER_W)])


@jax.jit
def _route(ids3, table_flat):
    mesh = plsc.VectorSubcoreMesh(
        core_axis_name="c", subcore_axis_name="s", num_cores=_NC,
        num_subcores=_NS,
    )
    call = pl.kernel(
        _router_body,
        out_type=jax.ShapeDtypeStruct((_BATCH, _NSB, _K, _CHUNK), jnp.int32),
        mesh=mesh,
        scratch_types=[
            pltpu.VMEM((_BLOCKS_PER_W, _CHUNK), jnp.int32),
            pltpu.VMEM((_BLOCKS_PER_W, _CHUNK), jnp.int32),
            pltpu.VMEM((_BLOCKS_PER_W, _K, _CHUNK), jnp.int32),
            pltpu.SemaphoreType.DMA,
        ],
        compiler_params=pltpu.CompilerParams(
            use_tc_tiling_on_sc=False, needs_layout_passes=False,
        ),
    )
    return call(ids3, table_flat)


def kernel(input, hash_table):
    ids3 = input.astype(jnp.int32).reshape(_BATCH, _NSB, _CHUNK).transpose(1, 0, 2)
    table_flat = hash_table.T.reshape(-1)
    out = _route(ids3, table_flat)
    return out.transpose(0, 1, 3, 2).reshape(_BATCH, _SEQ, _K)


# speedup vs baseline: 3.1410x; 1.0715x over previous
import jax
import jax.numpy as jnp
from jax import lax
from jax.experimental import pallas as pl
from jax.experimental.pallas import tpu as pltpu
from jax.experimental.pallas import tpu_sc as plsc

_VOCAB = 50257
_BATCH = 4
_SEQ = 4096
_K = 2
_NC = 2
_NS = 16
_NW = _NC * _NS
_L = 16
_CHUNK = 128
_NSB = _SEQ // _CHUNK
_BLOCKS_PER_W = _BATCH * _NSB // _NW


def _router_body(ids_hbm, table_hbm, out_hbm, ids_v, idx1_v, gout_v, sem):
    wid = lax.axis_index("s") * _NC + lax.axis_index("c")
    pltpu.sync_copy(ids_hbm.at[wid], ids_v)
    copies = [
        pltpu.async_copy(table_hbm.at[ids_v.at[j]], gout_v.at[j, 0], sem)
        for j in range(_BATCH)
    ]
    for j in range(_BATCH):
        for g in range(_CHUNK // _L):
            sl = pl.ds(g * _L, _L)
            idx1_v[j, sl] = ids_v[j, sl] + _VOCAB
    copies += [
        pltpu.async_copy(table_hbm.at[idx1_v.at[j]], gout_v.at[j, 1], sem)
        for j in range(_BATCH)
    ]
    for c in copies:
        c.wait()
    pltpu.sync_copy(gout_v, out_hbm.at[:, wid])


@jax.jit
def _route(ids3, table_flat):
    mesh = plsc.VectorSubcoreMesh(
        core_axis_name="c", subcore_axis_name="s", num_cores=_NC,
        num_subcores=_NS,
    )
    call = pl.kernel(
        _router_body,
        out_type=jax.ShapeDtypeStruct((_BATCH, _NSB, _K, _CHUNK), jnp.int32),
        mesh=mesh,
        scratch_types=[
            pltpu.VMEM((_BLOCKS_PER_W, _CHUNK), jnp.int32),
            pltpu.VMEM((_BLOCKS_PER_W, _CHUNK), jnp.int32),
            pltpu.VMEM((_BLOCKS_PER_W, _K, _CHUNK), jnp.int32),
            pltpu.SemaphoreType.DMA,
        ],
        compiler_params=pltpu.CompilerParams(
            use_tc_tiling_on_sc=False, needs_layout_passes=False,
        ),
    )
    return call(ids3, table_flat)


def kernel(input, hash_table):
    ids3 = input.astype(jnp.int32).reshape(_BATCH, _NSB, _CHUNK).transpose(1, 0, 2)
    table_flat = hash_table.T.reshape(-1)
    out = _route(ids3, table_flat)
    return out.transpose(0, 1, 3, 2).reshape(_BATCH, _SEQ, _K)
